# baseline (speedup 1.0000x reference)
import functools
import math

import jax
import jax.numpy as jnp
import numpy as np
from jax import lax
from jax.experimental import pallas as pl
from jax.experimental.pallas import tpu as pltpu
from jax.experimental.pallas import tpu_sc as plsc

VOCAB = 1000000
EMB_DIM = 64
BATCH = 4
SEQ = 2048
SCALE = math.sqrt(EMB_DIM)

NC, NS, L = 2, 16, 16
NW = NC * NS
B_TOTAL = BATCH * SEQ
B_PER_W = B_TOTAL // NW
PE_CHUNKS = SEQ // B_PER_W
BLK = 8
NBUF = 2
ROUND = 32
N_ROUNDS = B_PER_W // ROUND
PAIR_W = 2 * EMB_DIM


def _sinusoidal_pe(seq_len, d):
    pos = np.arange(seq_len, dtype=np.float32)[:, None]
    div = np.exp(np.arange(0, d, 2, dtype=np.float32) * (-math.log(10000.0) / d))
    pe = np.zeros((seq_len, d), dtype=np.float32)
    pe[:, 0::2] = np.sin(pos * div)
    pe[:, 1::2] = np.cos(pos * div)
    return pe

_PE2 = _sinusoidal_pe(SEQ, EMB_DIM).reshape(SEQ // 2, PAIR_W)


def _build_sc_kernel():
    mesh = plsc.VectorSubcoreMesh(core_axis_name="c", subcore_axis_name="s",
                                  num_cores=NC, num_subcores=NS)

    @functools.partial(
        pl.kernel,
        out_type=jax.ShapeDtypeStruct((B_TOTAL // 2, PAIR_W), jnp.float32),
        mesh=mesh,
        scratch_types=[
            pltpu.VMEM((B_PER_W + L,), jnp.int32),
            pltpu.VMEM((NBUF, ROUND, BLK, EMB_DIM), jnp.float32),
            pltpu.VMEM((B_PER_W // 2, PAIR_W), jnp.float32),
            pltpu.VMEM((B_PER_W // 2, PAIR_W), jnp.float32),
            pltpu.SemaphoreType.DMA,
            pltpu.SemaphoreType.DMA,
            pltpu.SemaphoreType.DMA,
        ],
    )
    def emb_kernel(idx_hbm, pe_hbm, tab_hbm, out_hbm,
                   idx_v, blk_v, pe_v, out_v, sem0, sem1, psem):
        sems = (sem0, sem1)
        wid = lax.axis_index("s") * NC + lax.axis_index("c")
        pltpu.sync_copy(idx_hbm.at[wid], idx_v.at[pl.ds(0, B_PER_W)])
        pe_base = lax.rem(wid, PE_CHUNKS) * (B_PER_W // 2)
        pe_copy = pltpu.async_copy(
            pe_hbm.at[pl.ds(pe_base, B_PER_W // 2)], pe_v, psem)

        def fire(r, buf):
            def enq(t, _):
                tok = idx_v[pl.ds(r * ROUND + t, L)][0]
                base = pl.multiple_of(
                    lax.shift_left(lax.shift_right_logical(tok, 3), 3), BLK)
                pltpu.async_copy(tab_hbm.at[pl.ds(base, BLK)],
                                 blk_v.at[buf, t], sems[buf])
                return _

            lax.fori_loop(0, ROUND, enq, None)

        def drain(buf):
            def dr(t, _):
                pltpu.make_async_copy(tab_hbm.at[pl.ds(0, BLK)],
                                      blk_v.at[0, 0], sems[buf]).wait()
                return _

            lax.fori_loop(0, ROUND, dr, None)

        def extract(r, buf):
            def body(t, _):
                j = r * ROUND + t
                tok = idx_v[pl.ds(j, L)][0]
                sub = lax.bitwise_and(tok, BLK - 1)
                half = lax.rem(j, 2)
                jrow = lax.div(j, 2)
                for d in range(EMB_DIM // L):
                    vals = blk_v[buf, t, sub, pl.ds(d * L, L)]
                    psl = pl.ds(half * EMB_DIM + d * L, L)
                    out_v[jrow, psl] = vals * SCALE + pe_v[jrow, psl]
                return _

            lax.fori_loop(0, ROUND, body, None)

        fire(0, 0)
        fire(1, 1)
        pe_copy.wait()
        for r in range(N_ROUNDS):
            drain(r % NBUF)
            extract(r, r % NBUF)
            if r + NBUF < N_ROUNDS:
                fire(r + NBUF, (r + NBUF) % NBUF)

        pltpu.sync_copy(
            out_v, out_hbm.at[pl.ds(wid * (B_PER_W // 2), B_PER_W // 2)])

    return emb_kernel


def kernel(x, tok_emb):
    idx = x.reshape(NW, B_PER_W).astype(jnp.int32)
    out2 = _build_sc_kernel()(idx, _PE2, tok_emb)
    return out2.reshape(BATCH, SEQ, EMB_DIM)


# speedup vs baseline: 1.4719x; 1.4719x over previous
import functools
import math

import jax
import jax.numpy as jnp
import numpy as np
from jax import lax
from jax.experimental import pallas as pl
from jax.experimental.pallas import tpu as pltpu
from jax.experimental.pallas import tpu_sc as plsc

VOCAB = 1000000
EMB_DIM = 64
BATCH = 4
SEQ = 2048
SCALE = math.sqrt(EMB_DIM)

NC, NS, L = 2, 16, 16
NW = NC * NS
B_TOTAL = BATCH * SEQ
B_PER_W = B_TOTAL // NW
PE_CHUNKS = SEQ // B_PER_W
BLK = 8
NBUF = 2
ROUND = 32
N_ROUNDS = B_PER_W // ROUND
PAIR_W = 2 * EMB_DIM


def _sinusoidal_pe(seq_len, d):
    pos = np.arange(seq_len, dtype=np.float32)[:, None]
    div = np.exp(np.arange(0, d, 2, dtype=np.float32) * (-math.log(10000.0) / d))
    pe = np.zeros((seq_len, d), dtype=np.float32)
    pe[:, 0::2] = np.sin(pos * div)
    pe[:, 1::2] = np.cos(pos * div)
    return pe

_PE2 = _sinusoidal_pe(SEQ, EMB_DIM).reshape(SEQ // 2, PAIR_W)


def _build_sc_kernel():
    mesh = plsc.VectorSubcoreMesh(core_axis_name="c", subcore_axis_name="s",
                                  num_cores=NC, num_subcores=NS)

    @functools.partial(
        pl.kernel,
        out_type=jax.ShapeDtypeStruct((B_TOTAL // 2, PAIR_W), jnp.float32),
        mesh=mesh,
        scratch_types=[
            pltpu.VMEM((B_PER_W + L,), jnp.int32),
            pltpu.VMEM((NBUF, ROUND, BLK, EMB_DIM), jnp.float32),
            pltpu.VMEM((B_PER_W // 2, PAIR_W), jnp.float32),
            pltpu.VMEM((B_PER_W // 2, PAIR_W), jnp.float32),
            pltpu.SemaphoreType.DMA,
            pltpu.SemaphoreType.DMA,
            pltpu.SemaphoreType.DMA,
        ],
    )
    def emb_kernel(idx_hbm, pe_hbm, tab_hbm, out_hbm,
                   idx_v, blk_v, pe_v, out_v, sem0, sem1, psem):
        sems = (sem0, sem1)
        wid = lax.axis_index("s") * NC + lax.axis_index("c")
        pltpu.sync_copy(idx_hbm.at[wid], idx_v.at[pl.ds(0, B_PER_W)])
        pe_base = lax.rem(wid, PE_CHUNKS) * (B_PER_W // 2)
        pe_copy = pltpu.async_copy(
            pe_hbm.at[pl.ds(pe_base, B_PER_W // 2)], pe_v, psem)

        def fire(r, buf):
            def enq(t, _):
                tok = idx_v[pl.ds(r * ROUND + t, L)][0]
                bid = lax.shift_right_logical(tok, 3)
                pltpu.async_copy(tab_hbm.at[bid], blk_v.at[buf, t], sems[buf])
                return _

            lax.fori_loop(0, ROUND, enq, None)

        def drain(buf):
            def dr(t, _):
                pltpu.make_async_copy(tab_hbm.at[0], blk_v.at[0, 0],
                                      sems[buf]).wait()
                return _

            lax.fori_loop(0, ROUND, dr, None)

        def extract(r, buf):
            def body(t, _):
                j = r * ROUND + t
                tok = idx_v[pl.ds(j, L)][0]
                sub = lax.bitwise_and(tok, BLK - 1)
                half = lax.rem(j, 2)
                jrow = lax.div(j, 2)
                for d in range(EMB_DIM // L):
                    vals = blk_v[buf, t, sub, pl.ds(d * L, L)]
                    psl = pl.ds(half * EMB_DIM + d * L, L)
                    out_v[jrow, psl] = vals * SCALE + pe_v[jrow, psl]
                return _

            lax.fori_loop(0, ROUND, body, None)

        fire(0, 0)
        fire(1, 1)
        pe_copy.wait()
        for r in range(N_ROUNDS):
            drain(r % NBUF)
            extract(r, r % NBUF)
            if r + NBUF < N_ROUNDS:
                fire(r + NBUF, (r + NBUF) % NBUF)

        pltpu.sync_copy(
            out_v, out_hbm.at[pl.ds(wid * (B_PER_W // 2), B_PER_W // 2)])

    return emb_kernel


def kernel(x, tok_emb):
    idx = x.reshape(NW, B_PER_W).astype(jnp.int32)
    table_blk = tok_emb.reshape(VOCAB // BLK, BLK, EMB_DIM)
    out2 = _build_sc_kernel()(idx, _PE2, table_blk)
    return out2.reshape(BATCH, SEQ, EMB_DIM)


# speedup vs baseline: 1.4736x; 1.0011x over previous
import functools
import math

import jax
import jax.numpy as jnp
import numpy as np
from jax import lax
from jax.experimental import pallas as pl
from jax.experimental.pallas import tpu as pltpu
from jax.experimental.pallas import tpu_sc as plsc

VOCAB = 1000000
EMB_DIM = 64
BATCH = 4
SEQ = 2048
SCALE = math.sqrt(EMB_DIM)

NC, NS, L = 2, 16, 16
NW = NC * NS
B_TOTAL = BATCH * SEQ
B_PER_W = B_TOTAL // NW
PE_CHUNKS = SEQ // B_PER_W
BLK = 8
NBUF = 2
ROUND = 32
N_ROUNDS = B_PER_W // ROUND
PAIR_W = 2 * EMB_DIM


def _sinusoidal_pe(seq_len, d):
    pos = np.arange(seq_len, dtype=np.float32)[:, None]
    div = np.exp(np.arange(0, d, 2, dtype=np.float32) * (-math.log(10000.0) / d))
    pe = np.zeros((seq_len, d), dtype=np.float32)
    pe[:, 0::2] = np.sin(pos * div)
    pe[:, 1::2] = np.cos(pos * div)
    return pe

_PE2 = _sinusoidal_pe(SEQ, EMB_DIM).reshape(SEQ // 2, PAIR_W)


def _build_sc_kernel():
    mesh = plsc.VectorSubcoreMesh(core_axis_name="c", subcore_axis_name="s",
                                  num_cores=NC, num_subcores=NS)

    @functools.partial(
        pl.kernel,
        out_type=jax.ShapeDtypeStruct((B_TOTAL // 2, PAIR_W), jnp.float32),
        mesh=mesh,
        scratch_types=[
            pltpu.VMEM((B_PER_W + L,), jnp.int32),
            pltpu.VMEM((NBUF, ROUND, BLK, EMB_DIM), jnp.float32),
            pltpu.VMEM((B_PER_W // 2, PAIR_W), jnp.float32),
            pltpu.VMEM((B_PER_W // 2, PAIR_W), jnp.float32),
            pltpu.SemaphoreType.DMA,
            pltpu.SemaphoreType.DMA,
            pltpu.SemaphoreType.DMA,
        ],
    )
    def emb_kernel(idx_hbm, pe_hbm, tab_hbm, out_hbm,
                   idx_v, blk_v, pe_v, out_v, sem0, sem1, psem):
        sems = (sem0, sem1)
        wid = lax.axis_index("s") * NC + lax.axis_index("c")
        pltpu.sync_copy(idx_hbm.at[wid], idx_v.at[pl.ds(0, B_PER_W)])
        pe_base = lax.rem(wid, PE_CHUNKS) * (B_PER_W // 2)
        pe_copy = pltpu.async_copy(
            pe_hbm.at[pl.ds(pe_base, B_PER_W // 2)], pe_v, psem)

        def fire(r, buf):
            for q in range(ROUND // L):
                bid16 = lax.shift_right_logical(
                    idx_v[pl.ds(r * ROUND + q * L, L)], 3)
                for k in range(L):
                    pltpu.async_copy(tab_hbm.at[bid16[k]],
                                     blk_v.at[buf, q * L + k], sems[buf])

        def drain(buf):
            pltpu.make_async_copy(tab_hbm.at[pl.ds(0, ROUND)], blk_v.at[buf],
                                  sems[buf]).wait()

        def extract(r, buf):
            def body(t, _):
                j = r * ROUND + t
                tok = idx_v[pl.ds(j, L)][0]
                sub = lax.bitwise_and(tok, BLK - 1)
                half = lax.rem(j, 2)
                jrow = lax.div(j, 2)
                for d in range(EMB_DIM // L):
                    vals = blk_v[buf, t, sub, pl.ds(d * L, L)]
                    psl = pl.ds(half * EMB_DIM + d * L, L)
                    out_v[jrow, psl] = vals * SCALE + pe_v[jrow, psl]
                return _

            lax.fori_loop(0, ROUND, body, None)

        fire(0, 0)
        fire(1, 1)
        pe_copy.wait()
        for r in range(N_ROUNDS):
            drain(r % NBUF)
            extract(r, r % NBUF)
            if r + NBUF < N_ROUNDS:
                fire(r + NBUF, (r + NBUF) % NBUF)

        pltpu.sync_copy(
            out_v, out_hbm.at[pl.ds(wid * (B_PER_W // 2), B_PER_W // 2)])

    return emb_kernel


def kernel(x, tok_emb):
    idx = x.reshape(NW, B_PER_W).astype(jnp.int32)
    table_blk = tok_emb.reshape(VOCAB // BLK, BLK, EMB_DIM)
    out2 = _build_sc_kernel()(idx, _PE2, table_blk)
    return out2.reshape(BATCH, SEQ, EMB_DIM)
